# baseline (speedup 1.0000x reference)
import functools

import jax
import jax.numpy as jnp
from jax import lax
from jax.experimental import pallas as pl
from jax.experimental.pallas import tpu as pltpu

_PAD = 0
_BOS = 2
_N = 2048
_C = 100000
_CB = 1536
_NBJ = pl.cdiv(_C, _CB)


def _dense_body(x_ref, t_ref, out_ref):
    j = pl.program_id(0)
    t = t_ref[...]
    nonpad = t != _PAD
    iota = lax.broadcasted_iota(jnp.int32, (_CB, _N), 0)
    match = iota == t - j * _CB
    gv = jnp.sum(jnp.where(match, x_ref[...], 0.0), axis=0, keepdims=True)
    gvm = jnp.where(nonpad, gv, 0.0)

    @pl.when(j == 0)
    def _():
        cs = (jnp.sum(x_ref[...], axis=0, keepdims=True)
              - x_ref[0:1, :] - x_ref[2:3, :])
        out_ref[0] = jnp.sum(jnp.where(nonpad, cs, 0.0))
        out_ref[1] = jnp.sum(gvm)
        out_ref[2] = jnp.sum(jnp.where(t == _BOS, gvm, 0.0))
        out_ref[3] = jnp.sum(jnp.where(nonpad, 1.0, 0.0))
        out_ref[4] = jnp.sum(jnp.where(t == _BOS, 1.0, 0.0))

    @pl.when(jnp.logical_and(j > 0, j < _NBJ - 1))
    def _():
        cs = jnp.sum(x_ref[...], axis=0, keepdims=True)
        out_ref[0] += jnp.sum(jnp.where(nonpad, cs, 0.0))
        out_ref[1] += jnp.sum(gvm)

    @pl.when(j == _NBJ - 1)
    def _():
        xm = jnp.where(iota < _C - j * _CB, x_ref[...], 0.0)
        cs = jnp.sum(xm, axis=0, keepdims=True)
        out_ref[0] += jnp.sum(jnp.where(nonpad, cs, 0.0))
        out_ref[1] += jnp.sum(gvm)


_dense_sums = pl.pallas_call(
    _dense_body,
    grid=(_NBJ,),
    in_specs=[
        pl.BlockSpec((_CB, _N), lambda j: (j, 0)),
        pl.BlockSpec((1, _N), lambda j: (0, 0)),
    ],
    out_specs=pl.BlockSpec(memory_space=pltpu.SMEM),
    out_shape=jax.ShapeDtypeStruct((5,), jnp.float32),
)


@jax.jit
def kernel(input, target, one_hot):
    t2d = target.reshape(1, _N).astype(jnp.int32)
    sums = _dense_sums(input.T, t2d)
    s_ex, g, g2, n_np, n_2 = sums[0], sums[1], sums[2], sums[3], sums[4]
    eps = one_hot[1]
    conf = 1.0 - eps * (_C - 2)
    loss = (n_np * conf * jnp.log(conf)
            + eps * jnp.log(eps) * ((_C - 3) * n_np + n_2)
            - (eps * s_ex - eps * (g - g2) + conf * g))
    nll = -g
    return loss, nll


# speedup vs baseline: 1.0040x; 1.0040x over previous
import functools

import jax
import jax.numpy as jnp
from jax import lax
from jax.experimental import pallas as pl
from jax.experimental.pallas import tpu as pltpu

_PAD = 0
_BOS = 2
_N = 2048
_C = 100000
_CB = 1000
_NBJ = pl.cdiv(_C, _CB)


def _dense_body(x_ref, t_ref, out_ref):
    j = pl.program_id(0)
    t = t_ref[...]
    nonpad = t != _PAD
    iota = lax.broadcasted_iota(jnp.int32, (_CB, _N), 0)
    match = iota == t - j * _CB
    gv = jnp.sum(jnp.where(match, x_ref[...], 0.0), axis=0, keepdims=True)
    gvm = jnp.where(nonpad, gv, 0.0)

    @pl.when(j == 0)
    def _():
        cs = (jnp.sum(x_ref[...], axis=0, keepdims=True)
              - x_ref[0:1, :] - x_ref[2:3, :])
        out_ref[0] = jnp.sum(jnp.where(nonpad, cs, 0.0))
        out_ref[1] = jnp.sum(gvm)
        out_ref[2] = jnp.sum(jnp.where(t == _BOS, gvm, 0.0))
        out_ref[3] = jnp.sum(jnp.where(nonpad, 1.0, 0.0))
        out_ref[4] = jnp.sum(jnp.where(t == _BOS, 1.0, 0.0))

    @pl.when(jnp.logical_and(j > 0, j < _NBJ - 1))
    def _():
        cs = jnp.sum(x_ref[...], axis=0, keepdims=True)
        out_ref[0] += jnp.sum(jnp.where(nonpad, cs, 0.0))
        out_ref[1] += jnp.sum(gvm)

    @pl.when(j == _NBJ - 1)
    def _():
        xm = jnp.where(iota < _C - j * _CB, x_ref[...], 0.0)
        cs = jnp.sum(xm, axis=0, keepdims=True)
        out_ref[0] += jnp.sum(jnp.where(nonpad, cs, 0.0))
        out_ref[1] += jnp.sum(gvm)


_dense_sums = pl.pallas_call(
    _dense_body,
    grid=(_NBJ,),
    in_specs=[
        pl.BlockSpec((_CB, _N), lambda j: (j, 0)),
        pl.BlockSpec((1, _N), lambda j: (0, 0)),
    ],
    out_specs=pl.BlockSpec(memory_space=pltpu.SMEM),
    out_shape=jax.ShapeDtypeStruct((5,), jnp.float32),
)


@jax.jit
def kernel(input, target, one_hot):
    t2d = target.reshape(1, _N).astype(jnp.int32)
    sums = _dense_sums(input.T, t2d)
    s_ex, g, g2, n_np, n_2 = sums[0], sums[1], sums[2], sums[3], sums[4]
    eps = one_hot[1]
    conf = 1.0 - eps * (_C - 2)
    loss = (n_np * conf * jnp.log(conf)
            + eps * jnp.log(eps) * ((_C - 3) * n_np + n_2)
            - (eps * s_ex - eps * (g - g2) + conf * g))
    nll = -g
    return loss, nll


# speedup vs baseline: 1.0276x; 1.0235x over previous
import functools

import jax
import jax.numpy as jnp
from jax import lax
from jax.experimental import pallas as pl
from jax.experimental.pallas import tpu as pltpu

_PAD = 0
_BOS = 2
_N = 2048
_C = 100000
_CB = 1280
_NBJ = pl.cdiv(_C, _CB)


def _dense_body(x_ref, t_ref, out_ref):
    j = pl.program_id(0)
    t = t_ref[...]
    nonpad = t != _PAD
    iota = lax.broadcasted_iota(jnp.int32, (_CB, _N), 0)
    match = iota == t - j * _CB
    gv = jnp.sum(jnp.where(match, x_ref[...], 0.0), axis=0, keepdims=True)
    gvm = jnp.where(nonpad, gv, 0.0)

    @pl.when(j == 0)
    def _():
        cs = (jnp.sum(x_ref[...], axis=0, keepdims=True)
              - x_ref[0:1, :] - x_ref[2:3, :])
        out_ref[0] = jnp.sum(jnp.where(nonpad, cs, 0.0))
        out_ref[1] = jnp.sum(gvm)
        out_ref[2] = jnp.sum(jnp.where(t == _BOS, gvm, 0.0))
        out_ref[3] = jnp.sum(jnp.where(nonpad, 1.0, 0.0))
        out_ref[4] = jnp.sum(jnp.where(t == _BOS, 1.0, 0.0))

    @pl.when(jnp.logical_and(j > 0, j < _NBJ - 1))
    def _():
        cs = jnp.sum(x_ref[...], axis=0, keepdims=True)
        out_ref[0] += jnp.sum(jnp.where(nonpad, cs, 0.0))
        out_ref[1] += jnp.sum(gvm)

    @pl.when(j == _NBJ - 1)
    def _():
        xm = jnp.where(iota < _C - j * _CB, x_ref[...], 0.0)
        cs = jnp.sum(xm, axis=0, keepdims=True)
        out_ref[0] += jnp.sum(jnp.where(nonpad, cs, 0.0))
        out_ref[1] += jnp.sum(gvm)


_dense_sums = pl.pallas_call(
    _dense_body,
    grid=(_NBJ,),
    in_specs=[
        pl.BlockSpec((_CB, _N), lambda j: (j, 0)),
        pl.BlockSpec((1, _N), lambda j: (0, 0)),
    ],
    out_specs=pl.BlockSpec(memory_space=pltpu.SMEM),
    out_shape=jax.ShapeDtypeStruct((5,), jnp.float32),
)


@jax.jit
def kernel(input, target, one_hot):
    t2d = target.reshape(1, _N).astype(jnp.int32)
    sums = _dense_sums(input.T, t2d)
    s_ex, g, g2, n_np, n_2 = sums[0], sums[1], sums[2], sums[3], sums[4]
    eps = one_hot[1]
    conf = 1.0 - eps * (_C - 2)
    loss = (n_np * conf * jnp.log(conf)
            + eps * jnp.log(eps) * ((_C - 3) * n_np + n_2)
            - (eps * s_ex - eps * (g - g2) + conf * g))
    nll = -g
    return loss, nll
